# baseline (speedup 1.0000x reference)
import functools

import jax
import jax.numpy as jnp
from jax import lax
from jax.experimental import pallas as pl
from jax.experimental.pallas import tpu as pltpu
from jax.experimental.pallas import tpu_sc as plsc

_B = 4
_S = 4096
_D = 1024
_NC = 2
_NS = 16
_NW = _NC * _NS
_WPB = _NW // _B
_BPW = _S // _WPB
_C = 32
_NCH = _BPW // _C
_NP = _NCH // 2

_mesh = plsc.VectorSubcoreMesh(core_axis_name="c", subcore_axis_name="s")


@functools.partial(
    pl.kernel,
    mesh=_mesh,
    out_type=jax.ShapeDtypeStruct((_B, _S, _D), jnp.float32),
    scratch_types=[
        pltpu.VMEM((_BPW,), jnp.int32),
        pltpu.VMEM((_C, _D), jnp.float32),
        pltpu.VMEM((_C, _D), jnp.float32),
        pltpu.SemaphoreType.DMA,
        pltpu.SemaphoreType.DMA,
    ],
)
def _gather(ids_hbm, table_hbm, out_hbm, idx_v, buf0, buf1, gsem0, gsem1):
    wid = lax.axis_index("s") * _NC + lax.axis_index("c")
    row = wid // _WPB
    col = (wid % _WPB) * _BPW
    pltpu.sync_copy(ids_hbm.at[row, pl.ds(col, _BPW)], idx_v)

    pltpu.async_copy(table_hbm.at[idx_v.at[pl.ds(0, _C)]], buf0, gsem0)

    def body(p, carry):
        i0 = 2 * p
        pltpu.async_copy(
            table_hbm.at[idx_v.at[pl.ds((i0 + 1) * _C, _C)]], buf1, gsem1
        )
        pltpu.make_async_copy(
            table_hbm.at[idx_v.at[pl.ds(i0 * _C, _C)]], buf0, gsem0
        ).wait()
        pltpu.sync_copy(buf0, out_hbm.at[row, pl.ds(col + i0 * _C, _C)])

        @pl.when(p < _NP - 1)
        def _():
            pltpu.async_copy(
                table_hbm.at[idx_v.at[pl.ds((i0 + 2) * _C, _C)]], buf0, gsem0
            )

        pltpu.make_async_copy(
            table_hbm.at[idx_v.at[pl.ds((i0 + 1) * _C, _C)]], buf1, gsem1
        ).wait()
        pltpu.sync_copy(buf1, out_hbm.at[row, pl.ds(col + (i0 + 1) * _C, _C)])
        return carry

    lax.fori_loop(0, _NP, body, 0)


@jax.jit
def kernel(input_ids, table):
    return _gather(input_ids, table)
